# baseline (speedup 1.0000x reference)
import functools

import jax
import jax.numpy as jnp
from jax import lax
from jax.experimental import pallas as pl
from jax.experimental.pallas import tpu as pltpu
from jax.experimental.pallas import tpu_sc as plsc

_CH = 1000


def _edge(valp_hbm, buf, ebuf, vq0, r0, lo, hi):
    @pl.when(hi > lo)
    def _do():
        ot = lax.div(lo, 8) * 8
        pltpu.sync_copy(valp_hbm.at[pl.ds(pl.multiple_of(vq0 + ot, 8), 8), :], ebuf)
        lanes = lax.iota(jnp.int32, 16)
        for i in range(7):
            r = lo + i

            @pl.when(r < hi)
            def _row(r=r):
                er = jnp.full((16,), r - ot, jnp.int32)
                dr = jnp.full((16,), r - r0, jnp.int32)
                for l in range(4):
                    x = plsc.load_gather(ebuf, [er, l * 16 + lanes])
                    plsc.store_scatter(buf, [dr, l * 16 + lanes], x)


def _patch(nw, w, valp_hbm, out_hbm, buf, ebuf, arc_row, arc_len, val_base, s08):
    c_first = lax.div(arc_row, _CH)
    c_last = lax.div(arc_row + jnp.maximum(arc_len, 1) - 1, _CH)
    c = c_first + lax.rem(lax.rem(w - c_first, nw) + nw, nw)

    @pl.when(jnp.logical_and(arc_len > 0, c <= c_last))
    def _do():
        r0 = c * _CH
        pltpu.sync_copy(out_hbm.at[pl.ds(pl.multiple_of(r0, 8), _CH), :], buf)
        a = jnp.maximum(arc_row, r0)
        b = jnp.minimum(arc_row + arc_len, r0 + _CH)
        vq0 = val_base + s08 - arc_row
        a8 = lax.div(a + 7, 8) * 8
        b8 = lax.div(b, 8) * 8
        li8 = lax.div(jnp.maximum(b8 - a8, 0), 8)
        off = jnp.int32(0)
        for k in reversed(range(7)):
            ln = 8 * (1 << k)
            bit = lax.rem(lax.div(li8, 1 << k), 2) == 1

            @pl.when(bit)
            def _seg(off=off, ln=ln):
                pltpu.sync_copy(
                    valp_hbm.at[pl.ds(pl.multiple_of(vq0 + a8 + off, 8), ln), :],
                    buf.at[pl.ds(pl.multiple_of(a8 - r0 + off, 8), ln), :],
                )

            off = off + jnp.where(bit, jnp.int32(ln), jnp.int32(0))
        _edge(valp_hbm, buf, ebuf, vq0, r0, a, jnp.minimum(a8, b))
        _edge(valp_hbm, buf, ebuf, vq0, r0, jnp.maximum(b8, a), b)
        pltpu.sync_copy(buf, out_hbm.at[pl.ds(pl.multiple_of(r0, 8), _CH), :])


def _sc_body(cap, size, nw, nc, mem_hbm, valp_hbm, svec_hbm, out_hbm, buf, ebuf, sv):
    w = lax.axis_index("s") * nc + lax.axis_index("c")
    nch = cap // _CH

    for t in range(-(-nch // nw)):
        c = w + nw * t

        @pl.when(c < nch)
        def _copy(c=c):
            pltpu.sync_copy(mem_hbm.at[pl.ds(pl.multiple_of(c * _CH, 8), _CH), :], buf)
            pltpu.sync_copy(buf, out_hbm.at[pl.ds(pl.multiple_of(c * _CH, 8), _CH), :])

    pltpu.sync_copy(svec_hbm, sv)
    s0 = lax.reduce_max(sv[...], axes=(0,))
    s08 = lax.rem(s0, 8)
    n1 = jnp.minimum(jnp.int32(size), cap - s0)
    patch = functools.partial(_patch, nw, w, valp_hbm, out_hbm, buf, ebuf)
    patch(s0, n1, jnp.int32(0), s08)
    patch(jnp.int32(0), jnp.int32(size) - n1, n1, s08)


def kernel(mem, val, store_index):
    cap, d = mem.shape
    size = min(val.shape[0], cap)
    assert cap % _CH == 0 and _CH % 8 == 0 and cap % 8 == 0

    info = plsc.get_sparse_core_info()
    nc, ns = info.num_cores, info.num_subcores
    nw = nc * ns
    assert size // _CH + 2 <= nw
    mesh = plsc.VectorSubcoreMesh(core_axis_name="c", subcore_axis_name="s")

    s0 = jnp.remainder(jnp.asarray(store_index, jnp.int32), cap)
    svec = jnp.full((16,), s0, dtype=jnp.int32)
    valp = lax.dynamic_update_slice(
        jnp.zeros((size + 8, d), val.dtype), val[:size], (lax.rem(s0, 8), 0)
    )

    body = functools.partial(_sc_body, cap, size, nw, nc)
    run = functools.partial(
        pl.kernel,
        mesh=mesh,
        out_type=jax.ShapeDtypeStruct((cap, d), mem.dtype),
        scratch_types=[
            pltpu.VMEM((_CH, d), jnp.float32),
            pltpu.VMEM((8, d), jnp.float32),
            pltpu.VMEM((16,), jnp.int32),
        ],
        compiler_params=pltpu.CompilerParams(needs_layout_passes=False),
    )(body)
    return run(mem, valp, svec)
